# baseline (speedup 1.0000x reference)
import functools

import jax
import jax.numpy as jnp
from jax import lax
from jax.experimental import pallas as pl
from jax.experimental.pallas import tpu as pltpu
from jax.experimental.pallas import tpu_sc as plsc

B = 16384
N_DENSE = 13
N_SPARSE = 26
VOCAB = 100000
EDIM = 16

NC = 2
NS = 16
NW = NC * NS
CHUNK = 128
TOT_IDX = B * N_SPARSE
TOT_CHUNKS = TOT_IDX // CHUNK
CHUNKS_PER_W = TOT_CHUNKS // NW
KB = 2
NBATCH = CHUNKS_PER_W // KB
BATCH_IDX = KB * CHUNK
STAGE_ROWS = BATCH_IDX * EDIM // 128

FROWS = 12504
TAB_ROWS = N_SPARSE * FROWS
OUT_ROWS = TOT_IDX * EDIM // 128

BLK = 512


RROWS = 4168
RSPLIT = FROWS // RROWS


def _repack_kernel(x_ref, p_ref, out_ref):
    acc = jnp.dot(x_ref[0], p_ref[0], preferred_element_type=jnp.float32)

    @pl.when(pl.program_id(2) == 0)
    def _():
        out_ref[...] = acc

    @pl.when(pl.program_id(2) != 0)
    def _():
        out_ref[...] += acc


def _repack(embed_tables, placement):
    grid = (N_SPARSE, RSPLIT, 8)
    return pl.pallas_call(
        _repack_kernel,
        grid=grid,
        in_specs=[
            pl.BlockSpec((1, RROWS, EDIM), lambda f, r, k: (f, k * RSPLIT + r, 0)),
            pl.BlockSpec((1, EDIM, 128), lambda f, r, k: (k, 0, 0)),
        ],
        out_specs=pl.BlockSpec((RROWS, 128), lambda f, r, k: (f * RSPLIT + r, 0)),
        out_shape=jax.ShapeDtypeStruct((TAB_ROWS, 128), jnp.float32),
        compiler_params=pltpu.CompilerParams(
            dimension_semantics=("arbitrary", "arbitrary", "arbitrary"),
        ),
    )(embed_tables, placement)


def _sc_gather(table128, wsp_flat, grow2d, goff2d, widx2d):
    mesh = plsc.VectorSubcoreMesh(core_axis_name="c", subcore_axis_name="s")

    @functools.partial(
        pl.kernel,
        out_type=[
            jax.ShapeDtypeStruct((OUT_ROWS, 128), jnp.float32),
            jax.ShapeDtypeStruct((TOT_CHUNKS, CHUNK), jnp.float32),
        ],
        mesh=mesh,
        compiler_params=pltpu.CompilerParams(use_tc_tiling_on_sc=False),
        scratch_types=[
            pltpu.VMEM((CHUNKS_PER_W, CHUNK), jnp.int32),
            pltpu.VMEM((CHUNKS_PER_W, CHUNK), jnp.int32),
            pltpu.VMEM((CHUNKS_PER_W, CHUNK), jnp.int32),
            pltpu.VMEM((2, BATCH_IDX, 128), jnp.float32),
            pltpu.VMEM((2, STAGE_ROWS, 128), jnp.float32),
            pltpu.VMEM((2, KB, CHUNK), jnp.float32),
            pltpu.SemaphoreType.DMA((2,)),
            pltpu.SemaphoreType.DMA((2,)),
        ],
    )
    def k(tab_hbm, wsp_hbm, grow_hbm, goff_hbm, widx_hbm, rows_out, wval_out,
          grow_v, goff_v, widx_v, buf, stage, wv, gsem, osem):
        wid = lax.axis_index("s") * NC + lax.axis_index("c")
        base = wid * CHUNKS_PER_W
        pltpu.sync_copy(grow_hbm.at[pl.ds(base, CHUNKS_PER_W)], grow_v)
        pltpu.sync_copy(goff_hbm.at[pl.ds(base, CHUNKS_PER_W)], goff_v)
        pltpu.sync_copy(widx_hbm.at[pl.ds(base, CHUNKS_PER_W)], widx_v)

        def gather_copies(s, p):
            cps = []
            for b in range(KB):
                cps.append(pltpu.make_async_copy(
                    tab_hbm.at[grow_v.at[s * KB + b]],
                    buf.at[p, pl.ds(b * CHUNK, CHUNK)], gsem.at[p]))
                cps.append(pltpu.make_async_copy(
                    wsp_hbm.at[widx_v.at[s * KB + b]],
                    wv.at[p, b], gsem.at[p]))
            return cps

        def out_copies(s, p):
            return [
                pltpu.make_async_copy(
                    stage.at[p],
                    rows_out.at[pl.ds((base + s * KB) * (CHUNK * EDIM // 128),
                                      STAGE_ROWS)],
                    osem.at[p]),
                pltpu.make_async_copy(
                    wv.at[p],
                    wval_out.at[pl.ds(base + s * KB, KB)],
                    osem.at[p]),
            ]

        def extract(s, p):
            def ebody(jj, carry):
                offs = goff_v[s * KB + lax.div(jj, 8),
                              pl.ds(lax.rem(jj, 8) * 16, 16)]
                for kk in range(16):
                    j = jj * 16 + kk
                    stage[p, jj * 2 + kk // 8, pl.ds((kk % 8) * EDIM, EDIM)] = \
                        buf[p, j, pl.ds(offs[kk], EDIM)]
                return carry
            lax.fori_loop(0, BATCH_IDX // 16, ebody, 0)

        for cp in gather_copies(0, 0):
            cp.start()
        for cp in gather_copies(1, 1):
            cp.start()

        def body(s, carry):
            p = lax.rem(s, 2)
            for cp in gather_copies(s, p):
                cp.wait()
            extract(s, p)
            for cp in out_copies(s, p):
                cp.start()

            @pl.when(s + 2 < NBATCH)
            def _():
                for cp in out_copies(s, p):
                    cp.wait()
                for cp in gather_copies(s + 2, p):
                    cp.start()
            return carry

        lax.fori_loop(0, NBATCH, body, 0)
        for tail in (NBATCH - 2, NBATCH - 1):
            for cp in out_copies(tail, tail % 2):
                cp.wait()

    return k(table128, wsp_flat, grow2d, goff2d, widx2d)


def _tc_mlp_kernel(dense_ref, emb_ref, wv_ref, w1d_ref, w1e_ref, b1_ref,
                   w2_ref, b2_ref, wf_ref, bf_ref, wd_ref, out_ref):
    x_d = dense_ref[...]
    x_e = emb_ref[...]
    h = x_d @ w1d_ref[...] + x_e @ w1e_ref[...] + b1_ref[...]
    h = jnp.maximum(h, 0.0)
    h = jnp.maximum(h @ w2_ref[...] + b2_ref[...], 0.0)
    deep = h @ wf_ref[...] + bf_ref[...]
    wide = x_d @ wd_ref[...] + jnp.sum(wv_ref[...], axis=1, keepdims=True)
    out_ref[...] = jax.nn.sigmoid(0.5 * (wide + deep))


def _tc_mlp(dense, emb, wvals, W1, b1, W2, b2, Wf, bf, w_dense):
    W1d = W1[:N_DENSE]
    W1e = W1[N_DENSE:]
    grid = (B // BLK,)
    const = lambda i: (0, 0)
    return pl.pallas_call(
        _tc_mlp_kernel,
        grid=grid,
        in_specs=[
            pl.BlockSpec((BLK, N_DENSE), lambda i: (i, 0)),
            pl.BlockSpec((BLK, N_SPARSE * EDIM), lambda i: (i, 0)),
            pl.BlockSpec((BLK, N_SPARSE), lambda i: (i, 0)),
            pl.BlockSpec((N_DENSE, 64), const),
            pl.BlockSpec((N_SPARSE * EDIM, 64), const),
            pl.BlockSpec((1, 64), const),
            pl.BlockSpec((64, 32), const),
            pl.BlockSpec((1, 32), const),
            pl.BlockSpec((32, 1), const),
            pl.BlockSpec((1, 1), const),
            pl.BlockSpec((N_DENSE, 1), const),
        ],
        out_specs=pl.BlockSpec((BLK, 1), lambda i: (i, 0)),
        out_shape=jax.ShapeDtypeStruct((B, 1), jnp.float32),
        compiler_params=pltpu.CompilerParams(
            dimension_semantics=("parallel",),
        ),
    )(dense, emb, wvals, W1d, W1e, b1.reshape(1, 64), W2, b2.reshape(1, 32),
      Wf, bf.reshape(1, 1), w_dense)


def kernel(inputs, embed_tables, w_sparse, w_dense, W1, b1, W2, b2, Wf, bf):
    dense = inputs[:, :N_DENSE]
    sparse_idx = inputs[:, N_DENSE:].astype(jnp.int32)
    widx = sparse_idx + (jnp.arange(N_SPARSE, dtype=jnp.int32) * VOCAB)[None, :]
    grow = (sparse_idx % FROWS) + (
        jnp.arange(N_SPARSE, dtype=jnp.int32) * FROWS)[None, :]
    grow2d = grow.reshape(TOT_CHUNKS, CHUNK)
    goff2d = ((sparse_idx // FROWS) * EDIM).reshape(TOT_CHUNKS, CHUNK)
    widx2d = widx.reshape(TOT_CHUNKS, CHUNK)

    lane = jnp.arange(128, dtype=jnp.int32)
    placement = (lane[None, None, :] ==
                 (EDIM * jnp.arange(8, dtype=jnp.int32)[:, None, None]
                  + jnp.arange(EDIM, dtype=jnp.int32)[None, :, None])
                 ).astype(jnp.float32)
    table128 = _repack(embed_tables, placement)
    wsp_flat = w_sparse.reshape(-1)

    rows128, wvals = _sc_gather(table128, wsp_flat, grow2d, goff2d, widx2d)
    emb = rows128.reshape(B, N_SPARSE * EDIM)
    wv = wvals.reshape(B, N_SPARSE)
    return _tc_mlp(dense, emb, wv, W1, b1, W2, b2, Wf, bf, w_dense)


# speedup vs baseline: 1.5154x; 1.5154x over previous
import functools

import jax
import jax.numpy as jnp
from jax import lax
from jax.experimental import pallas as pl
from jax.experimental.pallas import tpu as pltpu
from jax.experimental.pallas import tpu_sc as plsc

B = 16384
N_DENSE = 13
N_SPARSE = 26
VOCAB = 100000
EDIM = 16

NC = 2
NS = 16
NW = NC * NS
CHUNK = 128
TOT_IDX = B * N_SPARSE
TOT_CHUNKS = TOT_IDX // CHUNK
CHUNKS_PER_W = TOT_CHUNKS // NW
KB = 2
NBATCH = CHUNKS_PER_W // KB
BATCH_IDX = KB * CHUNK
STAGE_ROWS = BATCH_IDX * EDIM // 128

TAB_ROWS = N_SPARSE * VOCAB * EDIM // 128
OUT_ROWS = TOT_IDX * EDIM // 128

BLK = 512


def _sc_gather(table128, wsp_flat, grow2d, goff2d, widx2d):
    mesh = plsc.VectorSubcoreMesh(core_axis_name="c", subcore_axis_name="s")

    @functools.partial(
        pl.kernel,
        out_type=[
            jax.ShapeDtypeStruct((OUT_ROWS, 128), jnp.float32),
            jax.ShapeDtypeStruct((TOT_CHUNKS, CHUNK), jnp.float32),
        ],
        mesh=mesh,
        compiler_params=pltpu.CompilerParams(use_tc_tiling_on_sc=False),
        scratch_types=[
            pltpu.VMEM((CHUNKS_PER_W, CHUNK), jnp.int32),
            pltpu.VMEM((CHUNKS_PER_W, CHUNK), jnp.int32),
            pltpu.VMEM((CHUNKS_PER_W, CHUNK), jnp.int32),
            pltpu.VMEM((2, BATCH_IDX, 128), jnp.float32),
            pltpu.VMEM((2, STAGE_ROWS, 128), jnp.float32),
            pltpu.VMEM((2, KB, CHUNK), jnp.float32),
            pltpu.SemaphoreType.DMA((2,)),
            pltpu.SemaphoreType.DMA((2,)),
        ],
    )
    def k(tab_hbm, wsp_hbm, grow_hbm, goff_hbm, widx_hbm, rows_out, wval_out,
          grow_v, goff_v, widx_v, buf, stage, wv, gsem, osem):
        wid = lax.axis_index("s") * NC + lax.axis_index("c")
        base = wid * CHUNKS_PER_W
        pltpu.sync_copy(grow_hbm.at[pl.ds(base, CHUNKS_PER_W)], grow_v)
        pltpu.sync_copy(goff_hbm.at[pl.ds(base, CHUNKS_PER_W)], goff_v)
        pltpu.sync_copy(widx_hbm.at[pl.ds(base, CHUNKS_PER_W)], widx_v)

        def gather_copies(s, p):
            cps = []
            for b in range(KB):
                cps.append(pltpu.make_async_copy(
                    tab_hbm.at[grow_v.at[s * KB + b]],
                    buf.at[p, pl.ds(b * CHUNK, CHUNK)], gsem.at[p]))
                cps.append(pltpu.make_async_copy(
                    wsp_hbm.at[widx_v.at[s * KB + b]],
                    wv.at[p, b], gsem.at[p]))
            return cps

        def out_copies(s, p):
            return [
                pltpu.make_async_copy(
                    stage.at[p],
                    rows_out.at[pl.ds((base + s * KB) * (CHUNK * EDIM // 128),
                                      STAGE_ROWS)],
                    osem.at[p]),
                pltpu.make_async_copy(
                    wv.at[p],
                    wval_out.at[pl.ds(base + s * KB, KB)],
                    osem.at[p]),
            ]

        def extract(s, p):
            def ebody(jj, carry):
                offs = goff_v[s * KB + lax.div(jj, 8),
                              pl.ds(lax.rem(jj, 8) * 16, 16)]
                for kk in range(16):
                    j = jj * 16 + kk
                    stage[p, jj * 2 + kk // 8, pl.ds((kk % 8) * EDIM, EDIM)] = \
                        buf[p, j, pl.ds(offs[kk], EDIM)]
                return carry
            lax.fori_loop(0, BATCH_IDX // 16, ebody, 0)

        for cp in gather_copies(0, 0):
            cp.start()
        for cp in gather_copies(1, 1):
            cp.start()

        def body(s, carry):
            p = lax.rem(s, 2)
            for cp in gather_copies(s, p):
                cp.wait()
            extract(s, p)
            for cp in out_copies(s, p):
                cp.start()

            @pl.when(s + 2 < NBATCH)
            def _():
                for cp in out_copies(s, p):
                    cp.wait()
                for cp in gather_copies(s + 2, p):
                    cp.start()
            return carry

        lax.fori_loop(0, NBATCH, body, 0)
        for tail in (NBATCH - 2, NBATCH - 1):
            for cp in out_copies(tail, tail % 2):
                cp.wait()

    return k(table128, wsp_flat, grow2d, goff2d, widx2d)


def _tc_mlp_kernel(dense_ref, emb_ref, wv_ref, w1d_ref, w1e_ref, b1_ref,
                   w2_ref, b2_ref, wf_ref, bf_ref, wd_ref, out_ref):
    x_d = dense_ref[...]
    x_e = emb_ref[...]
    h = x_d @ w1d_ref[...] + x_e @ w1e_ref[...] + b1_ref[...]
    h = jnp.maximum(h, 0.0)
    h = jnp.maximum(h @ w2_ref[...] + b2_ref[...], 0.0)
    deep = h @ wf_ref[...] + bf_ref[...]
    wide = x_d @ wd_ref[...] + jnp.sum(wv_ref[...], axis=1, keepdims=True)
    out_ref[...] = jax.nn.sigmoid(0.5 * (wide + deep))


def _tc_mlp(dense, emb, wvals, W1, b1, W2, b2, Wf, bf, w_dense):
    W1d = W1[:N_DENSE]
    W1e = W1[N_DENSE:]
    grid = (B // BLK,)
    const = lambda i: (0, 0)
    return pl.pallas_call(
        _tc_mlp_kernel,
        grid=grid,
        in_specs=[
            pl.BlockSpec((BLK, N_DENSE), lambda i: (i, 0)),
            pl.BlockSpec((BLK, N_SPARSE * EDIM), lambda i: (i, 0)),
            pl.BlockSpec((BLK, N_SPARSE), lambda i: (i, 0)),
            pl.BlockSpec((N_DENSE, 64), const),
            pl.BlockSpec((N_SPARSE * EDIM, 64), const),
            pl.BlockSpec((1, 64), const),
            pl.BlockSpec((64, 32), const),
            pl.BlockSpec((1, 32), const),
            pl.BlockSpec((32, 1), const),
            pl.BlockSpec((1, 1), const),
            pl.BlockSpec((N_DENSE, 1), const),
        ],
        out_specs=pl.BlockSpec((BLK, 1), lambda i: (i, 0)),
        out_shape=jax.ShapeDtypeStruct((B, 1), jnp.float32),
        compiler_params=pltpu.CompilerParams(
            dimension_semantics=("parallel",),
        ),
    )(dense, emb, wvals, W1d, W1e, b1.reshape(1, 64), W2, b2.reshape(1, 32),
      Wf, bf.reshape(1, 1), w_dense)


def kernel(inputs, embed_tables, w_sparse, w_dense, W1, b1, W2, b2, Wf, bf):
    dense = inputs[:, :N_DENSE]
    sparse_idx = inputs[:, N_DENSE:].astype(jnp.int32)
    widx = sparse_idx + (jnp.arange(N_SPARSE, dtype=jnp.int32) * VOCAB)[None, :]
    grow2d = (widx >> 3).reshape(TOT_CHUNKS, CHUNK)
    goff2d = ((widx & 7) * EDIM).reshape(TOT_CHUNKS, CHUNK)
    widx2d = widx.reshape(TOT_CHUNKS, CHUNK)

    table128 = embed_tables[:, :VOCAB, :].reshape(TAB_ROWS, 128)
    wsp_flat = w_sparse.reshape(-1)

    rows128, wvals = _sc_gather(table128, wsp_flat, grow2d, goff2d, widx2d)
    emb = rows128.reshape(B, N_SPARSE * EDIM)
    wv = wvals.reshape(B, N_SPARSE)
    return _tc_mlp(dense, emb, wv, W1, b1, W2, b2, Wf, bf, w_dense)
